# baseline (speedup 1.0000x reference)
import functools

import jax
import jax.numpy as jnp
from jax import lax
from jax.experimental import pallas as pl
from jax.experimental.pallas import tpu as pltpu
from jax.experimental.pallas import tpu_sc as plsc

N_PRED = 100000
TBL_PAD = 100352
NC = 2
NS = 16
NW = NC * NS
LANES = 16
RH = 8
CW = 256
N_IN = 4
N_OUT = 2


def _sigmoid_body(x_ref, o_ref):
    o_ref[...] = jax.nn.sigmoid(x_ref[...])


def _bridge_body(tbl_hbm, idx_hbm, rl_hbm, kge_hbm, out_hbm,
                 tbl_v, idx_v, rl_v, kge_v, out_v,
                 in_sem, out_sem, n_per, ncg_shift):
    wid = lax.axis_index("s") * NC + lax.axis_index("c")
    base = wid * n_per
    ncg_mask = (1 << ncg_shift) - 1

    def slab(g):
        q = base + g
        return (q >> ncg_shift) * RH, (q & ncg_mask) * CW

    def start_in(g, slot):
        r0, c0 = slab(g)
        pltpu.async_copy(idx_hbm.at[pl.ds(r0, RH), pl.ds(c0, CW)],
                         idx_v.at[slot], in_sem.at[slot])
        pltpu.async_copy(rl_hbm.at[pl.ds(r0, RH), pl.ds(c0, CW)],
                         rl_v.at[slot], in_sem.at[slot])
        pltpu.async_copy(kge_hbm.at[pl.ds(r0, RH), pl.ds(c0, CW)],
                         kge_v.at[slot], in_sem.at[slot])

    def wait_in(slot):
        pltpu.make_async_copy(idx_hbm.at[pl.ds(0, RH), pl.ds(0, CW)],
                              idx_v.at[slot], in_sem.at[slot]).wait()
        pltpu.make_async_copy(rl_hbm.at[pl.ds(0, RH), pl.ds(0, CW)],
                              rl_v.at[slot], in_sem.at[slot]).wait()
        pltpu.make_async_copy(kge_hbm.at[pl.ds(0, RH), pl.ds(0, CW)],
                              kge_v.at[slot], in_sem.at[slot]).wait()

    def start_out(g, slot):
        r0, c0 = slab(g)
        pltpu.async_copy(out_v.at[slot],
                         out_hbm.at[pl.ds(r0, RH), pl.ds(c0, CW)],
                         out_sem.at[slot])

    def wait_out(slot):
        pltpu.make_async_copy(out_v.at[slot],
                              out_hbm.at[pl.ds(0, RH), pl.ds(0, CW)],
                              out_sem.at[slot]).wait()

    def compute(in_slot, out_slot):
        @plsc.parallel_loop(0, RH * CW, LANES, unroll=2)
        def _vec(o):
            r = o >> 8
            c = o & (CW - 1)
            iv = idx_v[in_slot, r, pl.ds(c, LANES)]
            logit = plsc.load_gather(tbl_v, [iv >> 7, iv & 127])
            rr = rl_v[in_slot, r, pl.ds(c, LANES)]
            kk = kge_v[in_slot, r, pl.ds(c, LANES)]
            out_v[out_slot, r, pl.ds(c, LANES)] = (
                kk + (rr - kk) / (1.0 + jnp.exp(-logit)))

    for s in range(N_IN - 1):
        start_in(s, s)
    pltpu.sync_copy(tbl_hbm, tbl_v)

    def chunk_body(g, carry):
        pre_g = g + N_IN - 1

        @pl.when(pre_g < n_per)
        def _():
            start_in(pre_g, pre_g & (N_IN - 1))

        wait_in(g & (N_IN - 1))

        @pl.when(g >= N_OUT)
        def _():
            wait_out(g & (N_OUT - 1))

        compute(g & (N_IN - 1), g & (N_OUT - 1))
        start_out(g, g & (N_OUT - 1))
        return carry

    lax.fori_loop(0, n_per, chunk_body, 0)
    for s in range(N_OUT):
        wait_out(s)


def kernel(rl_logprobs, kge_logprobs, pred_indices, alphas):
    B, K = rl_logprobs.shape
    assert K % RH == 0 and B % CW == 0
    ncg = B // CW
    ncg_shift = ncg.bit_length() - 1
    assert (1 << ncg_shift) == ncg
    n_chunks = (K // RH) * ncg
    n_per = n_chunks // NW
    assert n_per * NW == n_chunks

    tbl_2d = jnp.pad(alphas, (0, TBL_PAD - N_PRED)).reshape(
        TBL_PAD // 128, 128)

    idx_t = pred_indices.astype(jnp.int32).T
    rl_t = rl_logprobs.T
    kge_t = kge_logprobs.T

    body = functools.partial(_bridge_body, n_per=n_per, ncg_shift=ncg_shift)
    out_t = pl.kernel(
        body,
        out_type=jax.ShapeDtypeStruct((K, B), jnp.float32),
        mesh=plsc.VectorSubcoreMesh(
            core_axis_name="c", subcore_axis_name="s",
            num_cores=NC, num_subcores=NS),
        compiler_params=pltpu.CompilerParams(
            needs_layout_passes=False, use_tc_tiling_on_sc=True),
        scratch_types=[
            pltpu.VMEM((TBL_PAD // 128, 128), jnp.float32),
            pltpu.VMEM((N_IN, RH, CW), jnp.int32),
            pltpu.VMEM((N_IN, RH, CW), jnp.float32),
            pltpu.VMEM((N_IN, RH, CW), jnp.float32),
            pltpu.VMEM((N_OUT, RH, CW), jnp.float32),
            pltpu.SemaphoreType.DMA((N_IN,)),
            pltpu.SemaphoreType.DMA((N_OUT,)),
        ],
    )(tbl_2d, idx_t, rl_t, kge_t)
    return out_t.T


# speedup vs baseline: 1.0590x; 1.0590x over previous
import functools

import jax
import jax.numpy as jnp
from jax import lax
from jax.experimental import pallas as pl
from jax.experimental.pallas import tpu as pltpu
from jax.experimental.pallas import tpu_sc as plsc

N_PRED = 100000
TBL_PAD = 100352
NC = 2
NS = 16
NW = NC * NS
LANES = 16
RH = 8
CW = 256
N_IN = 8
N_OUT = 4


def _sigmoid_body(x_ref, o_ref):
    s = jax.nn.sigmoid(x_ref[...])
    b = jax.lax.bitcast_convert_type(s.reshape(TBL_PAD // 256, 2, 128),
                                     jnp.int32)
    o_ref[...] = ((b[:, 0, :] >> 16) & 0xFFFF) | (b[:, 1, :] & ~0xFFFF)


def _bridge_body(tbl_hbm, idx_hbm, rl_hbm, kge_hbm, out_hbm,
                 tbl_v, idx_v, rl_v, kge_v, out_v,
                 in_sem, out_sem, n_per, ncg_shift):
    wid = lax.axis_index("s") * NC + lax.axis_index("c")
    base = wid * n_per
    ncg_mask = (1 << ncg_shift) - 1

    def slab(g):
        q = base + g
        return (q >> ncg_shift) * RH, (q & ncg_mask) * CW

    def start_in(g, slot):
        r0, c0 = slab(g)
        pltpu.async_copy(idx_hbm.at[pl.ds(r0, RH), pl.ds(c0, CW)],
                         idx_v.at[slot], in_sem.at[slot])
        pltpu.async_copy(rl_hbm.at[pl.ds(r0, RH), pl.ds(c0, CW)],
                         rl_v.at[slot], in_sem.at[slot])
        pltpu.async_copy(kge_hbm.at[pl.ds(r0, RH), pl.ds(c0, CW)],
                         kge_v.at[slot], in_sem.at[slot])

    def wait_in(slot):
        pltpu.make_async_copy(idx_hbm.at[pl.ds(0, RH), pl.ds(0, CW)],
                              idx_v.at[slot], in_sem.at[slot]).wait()
        pltpu.make_async_copy(rl_hbm.at[pl.ds(0, RH), pl.ds(0, CW)],
                              rl_v.at[slot], in_sem.at[slot]).wait()
        pltpu.make_async_copy(kge_hbm.at[pl.ds(0, RH), pl.ds(0, CW)],
                              kge_v.at[slot], in_sem.at[slot]).wait()

    def start_out(g, slot):
        r0, c0 = slab(g)
        pltpu.async_copy(out_v.at[slot],
                         out_hbm.at[pl.ds(r0, RH), pl.ds(c0, CW)],
                         out_sem.at[slot])

    def wait_out(slot):
        pltpu.make_async_copy(out_v.at[slot],
                              out_hbm.at[pl.ds(0, RH), pl.ds(0, CW)],
                              out_sem.at[slot]).wait()

    def compute(in_slot, out_slot):
        @plsc.parallel_loop(0, RH * CW, LANES, unroll=2)
        def _vec(o):
            r = o >> 8
            c = o & (CW - 1)
            iv = idx_v[in_slot, r, pl.ds(c, LANES)]
            w = plsc.load_gather(tbl_v, [iv >> 8, iv & 127])
            sh = (1 - ((iv >> 7) & 1)) << 4
            a = plsc.bitcast((w << sh) & ~0xFFFF, jnp.float32)
            rr = rl_v[in_slot, r, pl.ds(c, LANES)]
            kk = kge_v[in_slot, r, pl.ds(c, LANES)]
            out_v[out_slot, r, pl.ds(c, LANES)] = kk + a * (rr - kk)

    for s in range(N_IN - 1):
        start_in(s, s)
    pltpu.sync_copy(tbl_hbm, tbl_v)

    def chunk_body(g, carry):
        pre_g = g + N_IN - 1

        @pl.when(pre_g < n_per)
        def _():
            start_in(pre_g, pre_g & (N_IN - 1))

        wait_in(g & (N_IN - 1))

        @pl.when(g >= N_OUT)
        def _():
            wait_out(g & (N_OUT - 1))

        compute(g & (N_IN - 1), g & (N_OUT - 1))
        start_out(g, g & (N_OUT - 1))
        return carry

    lax.fori_loop(0, n_per, chunk_body, 0)
    for s in range(N_OUT):
        wait_out(s)


def kernel(rl_logprobs, kge_logprobs, pred_indices, alphas):
    B, K = rl_logprobs.shape
    assert K % RH == 0 and B % CW == 0
    ncg = B // CW
    ncg_shift = ncg.bit_length() - 1
    assert (1 << ncg_shift) == ncg
    n_chunks = (K // RH) * ncg
    n_per = n_chunks // NW
    assert n_per * NW == n_chunks

    alphas_p = jnp.pad(alphas, (0, TBL_PAD - N_PRED))
    sig_tbl = pl.pallas_call(
        _sigmoid_body,
        out_shape=jax.ShapeDtypeStruct((TBL_PAD // 256, 128), jnp.int32),
    )(alphas_p.reshape(TBL_PAD // 128, 128))

    idx_t = pred_indices.astype(jnp.int32).T
    rl_t = rl_logprobs.T
    kge_t = kge_logprobs.T

    body = functools.partial(_bridge_body, n_per=n_per, ncg_shift=ncg_shift)
    out_t = pl.kernel(
        body,
        out_type=jax.ShapeDtypeStruct((K, B), jnp.float32),
        mesh=plsc.VectorSubcoreMesh(
            core_axis_name="c", subcore_axis_name="s",
            num_cores=NC, num_subcores=NS),
        compiler_params=pltpu.CompilerParams(
            needs_layout_passes=False, use_tc_tiling_on_sc=True),
        scratch_types=[
            pltpu.VMEM((TBL_PAD // 256, 128), jnp.int32),
            pltpu.VMEM((N_IN, RH, CW), jnp.int32),
            pltpu.VMEM((N_IN, RH, CW), jnp.float32),
            pltpu.VMEM((N_IN, RH, CW), jnp.float32),
            pltpu.VMEM((N_OUT, RH, CW), jnp.float32),
            pltpu.SemaphoreType.DMA((N_IN,)),
            pltpu.SemaphoreType.DMA((N_OUT,)),
        ],
    )(sig_tbl, idx_t, rl_t, kge_t)
    return out_t.T
